# Initial kernel scaffold; baseline (speedup 1.0000x reference)
import jax
import jax.numpy as jnp
from jax.experimental import pallas as pl


def kernel(primary, ss, x, y, z, W_primary, W_ss, gamma, beta):
    raise NotImplementedError("write your pallas kernel here")


# speedup vs baseline: 3.6974x; 3.6974x over previous
import functools

import jax
import jax.numpy as jnp
from jax import lax
from jax.experimental import pallas as pl
from jax.experimental.pallas import tpu as pltpu
from jax.experimental.pallas import tpu_sc as plsc

D = 128
LANES = 16
NCORES = 2
NSUB = 16
NW = NCORES * NSUB
TOK = 128
DSL = D // LANES


def _tanh(v):
    e = jnp.exp(v + v)
    return (e - 1.0) / (e + 1.0)


def _rsqrt_vec(v):
    i = lax.bitcast_convert_type(v, jnp.int32)
    i = 0x5F3759DF - lax.shift_right_logical(i, 1)
    r = lax.bitcast_convert_type(i, jnp.float32)
    h = 0.5 * v
    for _ in range(2):
        r = r * (1.5 - h * r * r)
    return r


def _allsum(v, perms):
    for p in perms:
        v = v + v.at[p].get(mode="promise_in_bounds")
    return v


def _tree_add(vs):
    vs = list(vs)
    while len(vs) > 1:
        vs = [vs[i] + vs[i + 1] for i in range(0, len(vs) - 1, 2)] + (
            [vs[-1]] if len(vs) % 2 else [])
    return vs[0]


def _make_sc_kernel(n_tokens):
    assert n_tokens % (NW * TOK) == 0
    tok_per_w = n_tokens // NW
    n_chunks = tok_per_w // TOK
    mesh = plsc.VectorSubcoreMesh(core_axis_name="c", subcore_axis_name="s")

    @functools.partial(
        pl.kernel,
        out_type=jax.ShapeDtypeStruct((n_tokens, D), jnp.float32),
        mesh=mesh,
        scratch_types=[
            pltpu.VMEM((TOK,), jnp.int32),
            pltpu.VMEM((TOK,), jnp.int32),
            pltpu.VMEM((TOK, D), jnp.float32),
            pltpu.VMEM((TOK, D), jnp.float32),
            pltpu.VMEM((TOK,), jnp.float32),
            pltpu.VMEM((TOK,), jnp.float32),
            pltpu.VMEM((TOK,), jnp.float32),
            pltpu.VMEM((TOK,), jnp.float32),
            pltpu.VMEM((TOK, D), jnp.float32),
            pltpu.VMEM((D,), jnp.float32),
            pltpu.VMEM((D,), jnp.float32),
            pltpu.SemaphoreType.DMA,
            pltpu.SemaphoreType.DMA,
        ],
    )
    def sc_embed(prim_hbm, ss_hbm, x_hbm, y_hbm, z_hbm, wp_hbm, ws_hbm,
                 gam_hbm, bet_hbm, out_hbm,
                 idx_p, idx_s, rows_p, rows_s, xv, yv, zv, gate_v, out_v,
                 gam_v, bet_v, sem_p, sem_s):
        wid = lax.axis_index("s") * NCORES + lax.axis_index("c")
        base = wid * tok_per_w
        pltpu.sync_copy(gam_hbm, gam_v)
        pltpu.sync_copy(bet_hbm, bet_v)
        gammas = [gam_v[pl.ds(LANES * j, LANES)] for j in range(DSL)]
        betas = [bet_v[pl.ds(LANES * j, LANES)] for j in range(DSL)]
        iota = lax.iota(jnp.int32, LANES)
        perms = [iota ^ k for k in (1, 2, 4, 8)]

        def chunk_body(c, _):
            tb = base + c * TOK
            pltpu.sync_copy(prim_hbm.at[pl.ds(tb, TOK)], idx_p)
            pltpu.sync_copy(ss_hbm.at[pl.ds(tb, TOK)], idx_s)
            cp_p = pltpu.async_copy(wp_hbm.at[idx_p], rows_p, sem_p)
            cp_s = pltpu.async_copy(ws_hbm.at[idx_s], rows_s, sem_s)
            pltpu.sync_copy(x_hbm.at[pl.ds(tb, TOK)], xv)
            pltpu.sync_copy(y_hbm.at[pl.ds(tb, TOK)], yv)
            pltpu.sync_copy(z_hbm.at[pl.ds(tb, TOK)], zv)
            for g in range(TOK // LANES):
                s = pl.ds(g * LANES, LANES)
                gate_v[s] = _tanh(xv[s]) * _tanh(yv[s]) * _tanh(zv[s])
            cp_p.wait()
            cp_s.wait()

            def grp_body(grp, _):
                gv = gate_v[pl.ds(grp * LANES, LANES)]
                for tt in range(LANES):
                    t = grp * LANES + tt
                    es = [rows_p[t, pl.ds(LANES * j, LANES)] +
                          rows_s[t, pl.ds(LANES * j, LANES)]
                          for j in range(DSL)]
                    s1 = _allsum(_tree_add(es), perms)
                    s2 = _allsum(_tree_add([e * e for e in es]), perms)
                    g = jnp.full((LANES,), gv[tt], dtype=jnp.float32)
                    mean = s1 * (1.0 / D)
                    var = s2 * (1.0 / D) - mean * mean
                    rstd = _rsqrt_vec(g * g * var + 1e-5)
                    a = rstd * g
                    cvec = a * mean
                    for j in range(DSL):
                        out_v[t, pl.ds(LANES * j, LANES)] = (
                            (es[j] * a - cvec) * gammas[j] + betas[j])
                return _

            lax.fori_loop(0, TOK // LANES, grp_body, None)
            pltpu.sync_copy(out_v, out_hbm.at[pl.ds(tb, TOK)])
            return _

        lax.fori_loop(0, n_chunks, chunk_body, None)

    return sc_embed


def kernel(primary, ss, x, y, z, W_primary, W_ss, gamma, beta):
    b, l = primary.shape
    n = b * l
    sc = _make_sc_kernel(n)
    out = sc(primary.reshape(n), ss.reshape(n),
             x.reshape(n), y.reshape(n), z.reshape(n),
             W_primary, W_ss, gamma, beta)
    return out.reshape(b, l, D)
